# baseline (speedup 1.0000x reference)
import jax
import jax.numpy as jnp
from jax.experimental import pallas as pl
from jax.experimental.pallas import tpu as pltpu

Q = 1024
D = 32
K = 100000
TOPK = 16
CHUNK = 4096
KPAD = 102400
PAD_VAL = 1e17


def _knn_body(q_ref, k_ref, y2_ref, ov_ref, oi_ref, bv_ref, bi_ref):
    j = pl.program_id(0)
    nchunks = pl.num_programs(0)

    @pl.when(j == 0)
    def _init():
        bv_ref[...] = jnp.full((Q, TOPK), jnp.inf, dtype=jnp.float32)
        bi_ref[...] = jnp.full((Q, TOPK), jnp.int32(2**31 - 1), dtype=jnp.int32)

    x = q_ref[...]
    x2 = jnp.sum(x * x, axis=1, keepdims=True)
    kc = k_ref[...]
    y2 = y2_ref[...]
    inner = -2.0 * jax.lax.dot_general(
        x, kc, (((1,), (1,)), ((), ())),
        preferred_element_type=jnp.float32)
    d = (x2 + inner) + y2

    col = jax.lax.broadcasted_iota(jnp.int32, (Q, CHUNK), 1)
    i16 = jax.lax.broadcasted_iota(jnp.int32, (Q, TOPK), 1)
    base = j * CHUNK

    bv = bv_ref[...]
    bi = bi_ref[...]

    def _cond(state):
        d_, m_, bv_, bi_ = state
        return jnp.any(m_ < bv_[:, TOPK - 1:TOPK])

    def _extract(state):
        d_, m_, bv_, bi_ = state
        mi = jnp.min(jnp.where(d_ == m_, col, CHUNK), axis=1,
                     keepdims=True)
        d_ = jnp.where(col == mi, jnp.inf, d_)
        gi = mi + base
        less = (bv_ < m_) | ((bv_ == m_) & (bi_ < gi))
        p = jnp.sum(less.astype(jnp.int32), axis=1, keepdims=True)
        bv_sh = jnp.concatenate([bv_[:, :1], bv_[:, :-1]], axis=1)
        bi_sh = jnp.concatenate([bi_[:, :1], bi_[:, :-1]], axis=1)
        bv_ = jnp.where(i16 < p, bv_, jnp.where(i16 == p, m_, bv_sh))
        bi_ = jnp.where(i16 < p, bi_, jnp.where(i16 == p, gi, bi_sh))
        m_ = jnp.min(d_, axis=1, keepdims=True)
        return d_, m_, bv_, bi_

    m0 = jnp.min(d, axis=1, keepdims=True)
    _, _, bv, bi = jax.lax.while_loop(_cond, _extract, (d, m0, bv, bi))
    bv_ref[...] = bv
    bi_ref[...] = bi

    @pl.when(j == nchunks - 1)
    def _emit():
        ov_ref[...] = -bv
        oi_ref[...] = bi


def _knn_call(queries, keys_pad, y2_pad):
    return pl.pallas_call(
        _knn_body,
        grid=(KPAD // CHUNK,),
        in_specs=[
            pl.BlockSpec((Q, D), lambda j: (0, 0)),
            pl.BlockSpec((CHUNK, D), lambda j: (j, 0)),
            pl.BlockSpec((1, CHUNK), lambda j: (0, j)),
        ],
        out_specs=[
            pl.BlockSpec((Q, TOPK), lambda j: (0, 0)),
            pl.BlockSpec((Q, TOPK), lambda j: (0, 0)),
        ],
        out_shape=[
            jax.ShapeDtypeStruct((Q, TOPK), jnp.float32),
            jax.ShapeDtypeStruct((Q, TOPK), jnp.int32),
        ],
        scratch_shapes=[
            pltpu.VMEM((Q, TOPK), jnp.float32),
            pltpu.VMEM((Q, TOPK), jnp.int32),
        ],
    )(queries, keys_pad, y2_pad)


def kernel(queries, keys, k):
    del k
    keys_pad = jnp.concatenate(
        [keys, jnp.full((KPAD - K, D), PAD_VAL, dtype=keys.dtype)], axis=0)
    y2_pad = jnp.sum(keys_pad * keys_pad, axis=-1, keepdims=True).T
    neg_vals, nn_idx = _knn_call(queries, keys_pad, y2_pad)
    center_idx = jnp.tile(
        jnp.arange(Q, dtype=nn_idx.dtype)[:, None], (1, TOPK))
    edge_index = jnp.stack((nn_idx, center_idx), axis=0)
    return neg_vals, edge_index


# speedup vs baseline: 1.2171x; 1.2171x over previous
import jax
import jax.numpy as jnp
from jax.experimental import pallas as pl
from jax.experimental.pallas import tpu as pltpu

Q = 1024
D = 32
K = 100000
TOPK = 16
CHUNK = 1024
KPAD = 102400
PAD_VAL = 1e17


def _knn_body(q_ref, k_ref, y2_ref, ov_ref, oi_ref, bv_ref, bi_ref):
    j = pl.program_id(0)
    nchunks = pl.num_programs(0)

    @pl.when(j == 0)
    def _init():
        bv_ref[...] = jnp.full((Q, TOPK), jnp.inf, dtype=jnp.float32)
        bi_ref[...] = jnp.full((Q, TOPK), jnp.int32(2**31 - 1), dtype=jnp.int32)

    x = q_ref[...]
    x2 = jnp.sum(x * x, axis=1, keepdims=True)
    kc = k_ref[...]
    y2 = y2_ref[...]
    inner = -2.0 * jax.lax.dot_general(
        x, kc, (((1,), (1,)), ((), ())),
        preferred_element_type=jnp.float32)
    d = (x2 + inner) + y2

    col = jax.lax.broadcasted_iota(jnp.int32, (Q, CHUNK), 1)
    i16 = jax.lax.broadcasted_iota(jnp.int32, (Q, TOPK), 1)
    base = j * CHUNK

    bv = bv_ref[...]
    bi = bi_ref[...]

    def _cond(state):
        d_, m_, bv_, bi_ = state
        return jnp.any(m_ < bv_[:, TOPK - 1:TOPK])

    def _extract(state):
        d_, m_, bv_, bi_ = state
        mi = jnp.min(jnp.where(d_ == m_, col, CHUNK), axis=1,
                     keepdims=True)
        d_ = jnp.where(col == mi, jnp.inf, d_)
        gi = mi + base
        less = (bv_ < m_) | ((bv_ == m_) & (bi_ < gi))
        p = jnp.sum(less.astype(jnp.int32), axis=1, keepdims=True)
        bv_sh = jnp.concatenate([bv_[:, :1], bv_[:, :-1]], axis=1)
        bi_sh = jnp.concatenate([bi_[:, :1], bi_[:, :-1]], axis=1)
        bv_ = jnp.where(i16 < p, bv_, jnp.where(i16 == p, m_, bv_sh))
        bi_ = jnp.where(i16 < p, bi_, jnp.where(i16 == p, gi, bi_sh))
        m_ = jnp.min(d_, axis=1, keepdims=True)
        return d_, m_, bv_, bi_

    m0 = jnp.min(d, axis=1, keepdims=True)
    _, _, bv, bi = jax.lax.while_loop(_cond, _extract, (d, m0, bv, bi))
    bv_ref[...] = bv
    bi_ref[...] = bi

    @pl.when(j == nchunks - 1)
    def _emit():
        ov_ref[...] = -bv
        oi_ref[...] = bi


def _knn_call(queries, keys_pad, y2_pad):
    return pl.pallas_call(
        _knn_body,
        grid=(KPAD // CHUNK,),
        in_specs=[
            pl.BlockSpec((Q, D), lambda j: (0, 0)),
            pl.BlockSpec((CHUNK, D), lambda j: (j, 0)),
            pl.BlockSpec((1, CHUNK), lambda j: (0, j)),
        ],
        out_specs=[
            pl.BlockSpec((Q, TOPK), lambda j: (0, 0)),
            pl.BlockSpec((Q, TOPK), lambda j: (0, 0)),
        ],
        out_shape=[
            jax.ShapeDtypeStruct((Q, TOPK), jnp.float32),
            jax.ShapeDtypeStruct((Q, TOPK), jnp.int32),
        ],
        scratch_shapes=[
            pltpu.VMEM((Q, TOPK), jnp.float32),
            pltpu.VMEM((Q, TOPK), jnp.int32),
        ],
    )(queries, keys_pad, y2_pad)


def kernel(queries, keys, k):
    del k
    keys_pad = jnp.concatenate(
        [keys, jnp.full((KPAD - K, D), PAD_VAL, dtype=keys.dtype)], axis=0)
    y2_pad = jnp.sum(keys_pad * keys_pad, axis=-1, keepdims=True).T
    neg_vals, nn_idx = _knn_call(queries, keys_pad, y2_pad)
    center_idx = jnp.tile(
        jnp.arange(Q, dtype=nn_idx.dtype)[:, None], (1, TOPK))
    edge_index = jnp.stack((nn_idx, center_idx), axis=0)
    return neg_vals, edge_index
